# baseline (speedup 1.0000x reference)
import functools

import jax
import jax.numpy as jnp
from jax import lax
from jax.experimental import pallas as pl
from jax.experimental.pallas import tpu as pltpu, tpu_sc as plsc

N = 50000
E = 1600000
D = 16
ROW = 128
RPG = 10
GRP = RPG * ROW
NGRP = E // GRP
NW = 32
NB = 2000
EPS_LN = 1e-5

NSL = 2
ESL = E // NSL
NGRP_SL = ESL // GRP
_GRP_BASE = NGRP_SL // NW
_GRP_EXTRA = NGRP_SL % NW
_NPAIR = (_GRP_BASE + 2) // 2
_NROWS_TILE = N // 16

RPG_S = 2
GRP_S = RPG_S * ROW
NGRP_S_SL = ESL // GRP_S
_GRP_S_BASE = NGRP_S_SL // NW
_GRP_S_EXTRA = NGRP_S_SL % NW
_NPAIR_S = (_GRP_S_BASE + 2) // 2


def _layer(X, W, b, g, be, act):
    Z = jnp.dot(X, W, preferred_element_type=jnp.float32) + b
    mu = jnp.mean(Z, axis=-1, keepdims=True)
    Zc = Z - mu
    var = jnp.mean(Zc * Zc, axis=-1, keepdims=True)
    Y = Zc * lax.rsqrt(var + EPS_LN) * g + be
    if act == 'silu':
        Y = Y * jax.nn.sigmoid(Y)
    elif act == 'tanh':
        Y = jnp.tanh(Y)
    return Y


def _enc_body(x_ref, W1, b1, g1, be1, W2, b2, g2, be2, o_ref):
    X = x_ref[...]
    A = _layer(X, W1[...], b1[...], g1[...], be1[...], 'silu')
    o_ref[...] = _layer(A, W2[...], b2[...], g2[...], be2[...], 'silu')


def _run_enc(x, p, interpret=False):
    (W1, b1, g1, be1), (W2, b2, g2, be2) = p
    params = [W1, b1.reshape(1, -1), g1.reshape(1, -1), be1.reshape(1, -1),
              W2, b2.reshape(1, -1), g2.reshape(1, -1), be2.reshape(1, -1)]
    in_specs = [pl.BlockSpec((NB, x.shape[1]), lambda i: (i, 0))]
    in_specs += [pl.BlockSpec(w.shape, lambda i: (0, 0)) for w in params]
    return pl.pallas_call(
        _enc_body,
        grid=(N // NB,),
        in_specs=in_specs,
        out_specs=pl.BlockSpec((NB, D), lambda i: (i, 0)),
        out_shape=jax.ShapeDtypeStruct((N, D), jnp.float32),
        interpret=interpret,
    )(x, *params)


def _gather_body(sl_ofs, h_hbm, s2d_hbm, e2d_hbm, hs_hbm, he_hbm, *refs):
    (idx_sa, idx_ea, idx_sb, idx_eb, buf_sa, buf_ea, buf_sb, buf_eb,
     sem_ia, sem_ib, sem_ga, sem_gb, sem_wa, sem_wb) = refs
    c = lax.axis_index("c")
    s = lax.axis_index("s")
    w = s * 2 + c
    g0 = w * _GRP_BASE + jnp.minimum(w, _GRP_EXTRA)
    cnt = _GRP_BASE + jnp.where(w < _GRP_EXTRA, 1, 0)

    def idx_cp(g, idx_s, idx_e, sem):
        a = pltpu.make_async_copy(
            s2d_hbm.at[pl.ds(RPG * (sl_ofs + g), RPG)], idx_s, sem)
        b = pltpu.make_async_copy(
            e2d_hbm.at[pl.ds(RPG * (sl_ofs + g), RPG)], idx_e, sem)
        return a, b

    def gath(idx_s, idx_e, buf_s, buf_e, sem):
        return ([pltpu.make_async_copy(h_hbm.at[idx_s.at[j]],
                                       buf_s.at[pl.ds(j * ROW, ROW)], sem)
                 for j in range(RPG)] +
                [pltpu.make_async_copy(h_hbm.at[idx_e.at[j]],
                                       buf_e.at[pl.ds(j * ROW, ROW)], sem)
                 for j in range(RPG)])

    def wb(g, buf_s, buf_e, sem):
        a = pltpu.make_async_copy(buf_s, hs_hbm.at[pl.ds(GRP * g, GRP)], sem)
        b = pltpu.make_async_copy(buf_e, he_hbm.at[pl.ds(GRP * g, GRP)], sem)
        return a, b

    @pl.when(cnt > 0)
    def _():
        for cp in idx_cp(g0, idx_sa, idx_ea, sem_ia):
            cp.start()

    def body(p, carry):
        ga = g0 + 2 * p
        gb = ga + 1
        na = 2 * p
        nb = 2 * p + 1

        @pl.when(nb < cnt)
        def _():
            for cp in idx_cp(gb, idx_sb, idx_eb, sem_ib):
                cp.start()

        @pl.when(na < cnt)
        def _():
            for cp in idx_cp(ga, idx_sa, idx_ea, sem_ia):
                cp.wait()

            @pl.when(p > 0)
            def _():
                for cp in wb(ga, buf_sa, buf_ea, sem_wa):
                    cp.wait()
            for cp in gath(idx_sa, idx_ea, buf_sa, buf_ea, sem_ga):
                cp.start()

        @pl.when(na < cnt)
        def _():
            for cp in gath(idx_sa, idx_ea, buf_sa, buf_ea, sem_ga):
                cp.wait()
            for cp in wb(ga, buf_sa, buf_ea, sem_wa):
                cp.start()

        @pl.when(na + 2 < cnt)
        def _():
            for cp in idx_cp(ga + 2, idx_sa, idx_ea, sem_ia):
                cp.start()

        @pl.when(nb < cnt)
        def _():
            for cp in idx_cp(gb, idx_sb, idx_eb, sem_ib):
                cp.wait()

            @pl.when(p > 0)
            def _():
                for cp in wb(gb, buf_sb, buf_eb, sem_wb):
                    cp.wait()
            for cp in gath(idx_sb, idx_eb, buf_sb, buf_eb, sem_gb):
                cp.start()
            for cp in gath(idx_sb, idx_eb, buf_sb, buf_eb, sem_gb):
                cp.wait()
            for cp in wb(gb, buf_sb, buf_eb, sem_wb):
                cp.start()

        return carry

    lax.fori_loop(0, _NPAIR, body, 0)

    @pl.when(cnt > 0)
    def _():
        for cp in wb(g0, buf_sa, buf_ea, sem_wa):
            cp.wait()

    @pl.when(cnt > 1)
    def _():
        for cp in wb(g0 + 1, buf_sb, buf_eb, sem_wb):
            cp.wait()


def _run_gather(h, s2d, e2d, sl):
    mesh = plsc.VectorSubcoreMesh(core_axis_name="c", subcore_axis_name="s")
    fn = pl.kernel(
        functools.partial(_gather_body, sl * NGRP_SL),
        out_type=(jax.ShapeDtypeStruct((ESL, D), jnp.float32),
                  jax.ShapeDtypeStruct((ESL, D), jnp.float32)),
        mesh=mesh,
        scratch_types=[
            pltpu.VMEM((RPG, ROW), jnp.int32),
            pltpu.VMEM((RPG, ROW), jnp.int32),
            pltpu.VMEM((RPG, ROW), jnp.int32),
            pltpu.VMEM((RPG, ROW), jnp.int32),
            pltpu.VMEM((GRP, D), jnp.float32),
            pltpu.VMEM((GRP, D), jnp.float32),
            pltpu.VMEM((GRP, D), jnp.float32),
            pltpu.VMEM((GRP, D), jnp.float32),
            pltpu.SemaphoreType.DMA,
            pltpu.SemaphoreType.DMA,
            pltpu.SemaphoreType.DMA,
            pltpu.SemaphoreType.DMA,
            pltpu.SemaphoreType.DMA,
            pltpu.SemaphoreType.DMA,
        ],
        compiler_params=pltpu.CompilerParams(use_tc_tiling_on_sc=False),
    )
    return fn(h, s2d, e2d)


EBR = 1000


def _edge_body(hs_ref, he_ref, W1a, W1b, b1, g1, be1, W2e, b2, g2, be2,
               B32, B17, P16, m16, s16, o_ref):
    Z = (jnp.dot(hs_ref[...], W1a[...], preferred_element_type=jnp.float32)
         + jnp.dot(he_ref[...], W1b[...], preferred_element_type=jnp.float32)
         + b1[...])
    mu = jnp.dot(Z, B32[...], preferred_element_type=jnp.float32)
    d = Z - mu
    var = jnp.dot(d * d, B32[...], preferred_element_type=jnp.float32)
    A = d * lax.rsqrt(var + EPS_LN) * g1[...] + be1[...]
    A = A * jax.nn.sigmoid(A)
    Z2 = jnp.dot(A, W2e[...], preferred_element_type=jnp.float32) + b2[...]
    mu2 = jnp.dot(Z2, B17[...], preferred_element_type=jnp.float32)
    d2 = Z2 - mu2
    var2 = jnp.dot(d2 * d2, B17[...], preferred_element_type=jnp.float32)
    Y = d2 * lax.rsqrt(var2 + EPS_LN) * g2[...] + be2[...]
    Y = Y * jax.nn.sigmoid(Y)
    expw = jnp.exp(jnp.dot(Y, P16[...], preferred_element_type=jnp.float32))
    o_ref[...] = (Y * m16[...] + s16[...]) * expw


def _edge_setup(p):
    (W1, b1, g1, be1), (W2, b2, g2, be2) = p
    I8 = jnp.eye(8, dtype=jnp.float32)
    pos = jnp.arange(32)
    W2p = jnp.pad(W2, ((0, 0), (0, 15)))
    seg17 = jnp.where(pos < 17, 1.0 / 17.0, 0.0)[:, None] * jnp.ones((1, 32))
    p16 = jnp.where(pos == 16, 1.0, 0.0)[:, None] * jnp.ones((1, 32))
    pad17 = lambda v: jnp.pad(v, (0, 15))
    return [
        jnp.kron(I8, W1[:D]), jnp.kron(I8, W1[D:]),
        jnp.tile(b1, 8).reshape(1, -1), jnp.tile(g1, 8).reshape(1, -1),
        jnp.tile(be1, 8).reshape(1, -1),
        jnp.kron(I8, W2p),
        jnp.tile(pad17(b2), 8).reshape(1, -1),
        jnp.tile(pad17(g2), 8).reshape(1, -1),
        jnp.tile(pad17(be2), 8).reshape(1, -1),
        jnp.kron(I8, jnp.full((32, 32), 1.0 / 32.0)),
        jnp.kron(I8, seg17),
        jnp.kron(I8, p16),
        jnp.tile(jnp.where(pos < D, 1.0, 0.0), 8).reshape(1, -1),
        jnp.tile(jnp.where(pos == D, 1.0, 0.0), 8).reshape(1, -1),
    ]


def _run_edge(hs_pk, he_pk, params, interpret=False):
    in_specs = [pl.BlockSpec((EBR, 128), lambda i: (i, 0)),
                pl.BlockSpec((EBR, 128), lambda i: (i, 0))]
    in_specs += [pl.BlockSpec(w.shape, lambda i: (0,) * w.ndim) for w in params]
    n_pk = hs_pk.shape[0]
    return pl.pallas_call(
        _edge_body,
        grid=(n_pk // EBR,),
        in_specs=in_specs,
        out_specs=pl.BlockSpec((EBR, 256), lambda i: (i, 0)),
        out_shape=jax.ShapeDtypeStruct((n_pk, 256), jnp.float32),
        interpret=interpret,
    )(hs_pk, he_pk, *params)


def _scatter_body(sl, ew_hbm, e2d_hbm, zeros_hbm, acc_hbm, *refs):
    (idx_a, idx_b, buf_a, buf_b, sem_a, sem_b, sem_z, acc) = refs
    c = lax.axis_index("c")
    s = lax.axis_index("s")
    w = s * 2 + c
    g0 = w * _GRP_S_BASE + jnp.minimum(w, _GRP_S_EXTRA)
    cnt = _GRP_S_BASE + jnp.where(w < _GRP_S_EXTRA, 1, 0)
    idx_base = sl * NGRP_S_SL

    def fetch(g, idx, buf, sem):
        return (pltpu.make_async_copy(
                    ew_hbm.at[pl.ds(GRP_S * g, GRP_S)], buf, sem),
                pltpu.make_async_copy(
                    e2d_hbm.at[pl.ds(RPG_S * (idx_base + g), RPG_S)],
                    idx, sem))

    def scat(idx, buf):
        for j in range(RPG_S):
            pltpu.sync_copy(buf.at[pl.ds(j * ROW, ROW)],
                            acc.at[idx.at[j]], add=True)

    zcp = pltpu.make_async_copy(
        zeros_hbm.at[pl.ds(s * _NROWS_TILE, _NROWS_TILE)],
        acc.at[pl.ds(s * _NROWS_TILE, _NROWS_TILE)], sem_z)
    zcp.start()
    for cp in fetch(g0, idx_a, buf_a, sem_a):
        cp.start()
    zcp.wait()
    plsc.subcore_barrier()

    def body(p, carry):
        ga = g0 + 2 * p
        gb = ga + 1

        @pl.when(2 * p < cnt)
        def _():
            for cp in fetch(ga, idx_a, buf_a, sem_a):
                cp.wait()

            @pl.when(2 * p + 1 < cnt)
            def _():
                for cp in fetch(gb, idx_b, buf_b, sem_b):
                    cp.start()
            scat(idx_a, buf_a)

            @pl.when(2 * p + 2 < cnt)
            def _():
                for cp in fetch(ga + 2, idx_a, buf_a, sem_a):
                    cp.start()

        @pl.when(2 * p + 1 < cnt)
        def _():
            for cp in fetch(gb, idx_b, buf_b, sem_b):
                cp.wait()
            scat(idx_b, buf_b)
        return carry

    lax.fori_loop(0, _NPAIR_S, body, 0)
    plsc.subcore_barrier()
    pltpu.sync_copy(acc.at[pl.ds(s * _NROWS_TILE, _NROWS_TILE)],
                    acc_hbm.at[c, pl.ds(s * _NROWS_TILE, _NROWS_TILE)])


def _run_scatter(ew, e2d, zeros, sl):
    mesh = plsc.VectorSubcoreMesh(core_axis_name="c", subcore_axis_name="s")
    fn = pl.kernel(
        functools.partial(_scatter_body, sl),
        out_type=jax.ShapeDtypeStruct((2, N, 32), jnp.float32),
        mesh=mesh,
        scratch_types=[
            pltpu.VMEM((RPG_S, ROW), jnp.int32),
            pltpu.VMEM((RPG_S, ROW), jnp.int32),
            pltpu.VMEM((GRP_S, 32), jnp.float32),
            pltpu.VMEM((GRP_S, 32), jnp.float32),
            pltpu.SemaphoreType.DMA,
            pltpu.SemaphoreType.DMA,
            pltpu.SemaphoreType.DMA,
            pltpu.VMEM_SHARED((N, 32), jnp.float32),
        ],
        compiler_params=pltpu.CompilerParams(use_tc_tiling_on_sc=False),
    )
    return fn(ew, e2d, zeros)


def _node_body(h_ref, acc_ref, acc1_ref,
               Wn1a, Wn1b, bn1, gn1, ben1, Wn2, bn2, gn2, ben2,
               Wd1, bd1, gd1, bed1, Wd2, bd2, gd2, bed2, o_ref):
    accs = acc_ref[0] + acc_ref[1] + acc1_ref[0] + acc1_ref[1]
    agg = accs[:, :D] / (accs[:, D:D + 1] + 1e-16)
    Z = (jnp.dot(h_ref[...], Wn1a[...], preferred_element_type=jnp.float32)
         + jnp.dot(agg, Wn1b[...], preferred_element_type=jnp.float32)
         + bn1[...])
    mu = jnp.mean(Z, axis=-1, keepdims=True)
    Zc = Z - mu
    var = jnp.mean(Zc * Zc, axis=-1, keepdims=True)
    A = Zc * lax.rsqrt(var + EPS_LN) * gn1[...] + ben1[...]
    A = A * jax.nn.sigmoid(A)
    h3 = _layer(A, Wn2[...], bn2[...], gn2[...], ben2[...], 'silu')
    B1 = _layer(h3, Wd1[...], bd1[...], gd1[...], bed1[...], 'silu')
    T = _layer(B1, Wd2[...], bd2[...], gd2[...], bed2[...], 'tanh')
    nrm = jnp.sqrt(jnp.sum(T * T, axis=-1, keepdims=True)) + 1e-12
    o_ref[...] = T / nrm


def _run_node(h, acc, acc1, node_p, dec1_p, interpret=False):
    (Wn1, bn1, gn1, ben1), (Wn2, bn2, gn2, ben2) = node_p
    (Wd1, bd1, gd1, bed1), (Wd2, bd2, gd2, bed2) = dec1_p
    params = [Wn1[:D], Wn1[D:], bn1.reshape(1, -1), gn1.reshape(1, -1),
              ben1.reshape(1, -1), Wn2, bn2.reshape(1, -1), gn2.reshape(1, -1),
              ben2.reshape(1, -1),
              Wd1, bd1.reshape(1, -1), gd1.reshape(1, -1), bed1.reshape(1, -1),
              Wd2, bd2.reshape(1, -1), gd2.reshape(1, -1), bed2.reshape(1, -1)]
    in_specs = [pl.BlockSpec((NB, D), lambda i: (i, 0)),
                pl.BlockSpec((2, NB, 32), lambda i: (0, i, 0)),
                pl.BlockSpec((2, NB, 32), lambda i: (0, i, 0))]
    in_specs += [pl.BlockSpec(w.shape, lambda i: (0, 0)) for w in params]
    return pl.pallas_call(
        _node_body,
        grid=(N // NB,),
        in_specs=in_specs,
        out_specs=pl.BlockSpec((NB, 8), lambda i: (i, 0)),
        out_shape=jax.ShapeDtypeStruct((N, 8), jnp.float32),
        interpret=interpret,
    )(h, acc, acc1, *params)


def kernel(x, start, end, enc_p, net0_p, edge_p, node_p, dec0_p, dec1_p):
    del net0_p, dec0_p
    s2d = start.reshape(E // ROW, ROW)
    e2d = end.reshape(E // ROW, ROW)
    zeros = jnp.zeros((N, 32), jnp.float32)
    h = _run_enc(x, enc_p)
    eparams = _edge_setup(edge_p)
    hs0, he0 = _run_gather(h, s2d, e2d, 0)
    hs1, he1 = _run_gather(h, s2d, e2d, 1)
    ew0 = _run_edge(hs0.reshape(ESL // 8, 128), he0.reshape(ESL // 8, 128),
                    eparams)
    ew1 = _run_edge(hs1.reshape(ESL // 8, 128), he1.reshape(ESL // 8, 128),
                    eparams)
    acc0 = _run_scatter(ew0.reshape(ESL, 32), e2d, zeros, 0)
    acc1 = _run_scatter(ew1.reshape(ESL, 32), e2d, zeros, 1)
    return _run_node(h, acc0, acc1, node_p, dec1_p)


# speedup vs baseline: 1.0276x; 1.0276x over previous
import functools

import jax
import jax.numpy as jnp
from jax import lax
from jax.experimental import pallas as pl
from jax.experimental.pallas import tpu as pltpu, tpu_sc as plsc

N = 50000
E = 1600000
D = 16
ROW = 128
RPG = 10
GRP = RPG * ROW
NGRP = E // GRP
NW = 32
NB = 2000
EPS_LN = 1e-5

NSL = 2
ESL = E // NSL
NGRP_SL = ESL // GRP
_GRP_BASE = NGRP_SL // NW
_GRP_EXTRA = NGRP_SL % NW
_NPAIR = (_GRP_BASE + 2) // 2
_NROWS_TILE = N // 16

RPG_S = 2
GRP_S = RPG_S * ROW
NGRP_S_SL = ESL // GRP_S
_NW_S = NW // NSL
_GRP_S_BASE = NGRP_S_SL // _NW_S
_GRP_S_EXTRA = NGRP_S_SL % _NW_S
_NPAIR_S = (_GRP_S_BASE + 2) // 2


def _layer(X, W, b, g, be, act):
    Z = jnp.dot(X, W, preferred_element_type=jnp.float32) + b
    mu = jnp.mean(Z, axis=-1, keepdims=True)
    Zc = Z - mu
    var = jnp.mean(Zc * Zc, axis=-1, keepdims=True)
    Y = Zc * lax.rsqrt(var + EPS_LN) * g + be
    if act == 'silu':
        Y = Y * jax.nn.sigmoid(Y)
    elif act == 'tanh':
        Y = jnp.tanh(Y)
    return Y


def _enc_body(x_ref, W1, b1, g1, be1, W2, b2, g2, be2, o_ref):
    X = x_ref[...]
    A = _layer(X, W1[...], b1[...], g1[...], be1[...], 'silu')
    o_ref[...] = _layer(A, W2[...], b2[...], g2[...], be2[...], 'silu')


def _run_enc(x, p, interpret=False):
    (W1, b1, g1, be1), (W2, b2, g2, be2) = p
    params = [W1, b1.reshape(1, -1), g1.reshape(1, -1), be1.reshape(1, -1),
              W2, b2.reshape(1, -1), g2.reshape(1, -1), be2.reshape(1, -1)]
    in_specs = [pl.BlockSpec((NB, x.shape[1]), lambda i: (i, 0))]
    in_specs += [pl.BlockSpec(w.shape, lambda i: (0, 0)) for w in params]
    return pl.pallas_call(
        _enc_body,
        grid=(N // NB,),
        in_specs=in_specs,
        out_specs=pl.BlockSpec((NB, D), lambda i: (i, 0)),
        out_shape=jax.ShapeDtypeStruct((N, D), jnp.float32),
        interpret=interpret,
    )(x, *params)


def _gather_body(sl_ofs, h_hbm, s2d_hbm, e2d_hbm, hs_hbm, he_hbm, *refs):
    (idx_sa, idx_ea, idx_sb, idx_eb, buf_sa, buf_ea, buf_sb, buf_eb,
     sem_ia, sem_ib, sem_ga, sem_gb, sem_wa, sem_wb) = refs
    c = lax.axis_index("c")
    s = lax.axis_index("s")
    w = s * 2 + c
    g0 = w * _GRP_BASE + jnp.minimum(w, _GRP_EXTRA)
    cnt = _GRP_BASE + jnp.where(w < _GRP_EXTRA, 1, 0)

    def idx_cp(g, idx_s, idx_e, sem):
        a = pltpu.make_async_copy(
            s2d_hbm.at[pl.ds(RPG * (sl_ofs + g), RPG)], idx_s, sem)
        b = pltpu.make_async_copy(
            e2d_hbm.at[pl.ds(RPG * (sl_ofs + g), RPG)], idx_e, sem)
        return a, b

    def gath(idx_s, idx_e, buf_s, buf_e, sem):
        return ([pltpu.make_async_copy(h_hbm.at[idx_s.at[j]],
                                       buf_s.at[pl.ds(j * ROW, ROW)], sem)
                 for j in range(RPG)] +
                [pltpu.make_async_copy(h_hbm.at[idx_e.at[j]],
                                       buf_e.at[pl.ds(j * ROW, ROW)], sem)
                 for j in range(RPG)])

    def wb(g, buf_s, buf_e, sem):
        a = pltpu.make_async_copy(buf_s, hs_hbm.at[pl.ds(GRP * g, GRP)], sem)
        b = pltpu.make_async_copy(buf_e, he_hbm.at[pl.ds(GRP * g, GRP)], sem)
        return a, b

    @pl.when(cnt > 0)
    def _():
        for cp in idx_cp(g0, idx_sa, idx_ea, sem_ia):
            cp.start()

    def body(p, carry):
        ga = g0 + 2 * p
        gb = ga + 1
        na = 2 * p
        nb = 2 * p + 1

        @pl.when(nb < cnt)
        def _():
            for cp in idx_cp(gb, idx_sb, idx_eb, sem_ib):
                cp.start()

        @pl.when(na < cnt)
        def _():
            for cp in idx_cp(ga, idx_sa, idx_ea, sem_ia):
                cp.wait()

            @pl.when(p > 0)
            def _():
                for cp in wb(ga, buf_sa, buf_ea, sem_wa):
                    cp.wait()
            for cp in gath(idx_sa, idx_ea, buf_sa, buf_ea, sem_ga):
                cp.start()

        @pl.when(na < cnt)
        def _():
            for cp in gath(idx_sa, idx_ea, buf_sa, buf_ea, sem_ga):
                cp.wait()
            for cp in wb(ga, buf_sa, buf_ea, sem_wa):
                cp.start()

        @pl.when(na + 2 < cnt)
        def _():
            for cp in idx_cp(ga + 2, idx_sa, idx_ea, sem_ia):
                cp.start()

        @pl.when(nb < cnt)
        def _():
            for cp in idx_cp(gb, idx_sb, idx_eb, sem_ib):
                cp.wait()

            @pl.when(p > 0)
            def _():
                for cp in wb(gb, buf_sb, buf_eb, sem_wb):
                    cp.wait()
            for cp in gath(idx_sb, idx_eb, buf_sb, buf_eb, sem_gb):
                cp.start()
            for cp in gath(idx_sb, idx_eb, buf_sb, buf_eb, sem_gb):
                cp.wait()
            for cp in wb(gb, buf_sb, buf_eb, sem_wb):
                cp.start()

        return carry

    lax.fori_loop(0, _NPAIR, body, 0)

    @pl.when(cnt > 0)
    def _():
        for cp in wb(g0, buf_sa, buf_ea, sem_wa):
            cp.wait()

    @pl.when(cnt > 1)
    def _():
        for cp in wb(g0 + 1, buf_sb, buf_eb, sem_wb):
            cp.wait()


def _run_gather(h, s2d, e2d, sl):
    mesh = plsc.VectorSubcoreMesh(core_axis_name="c", subcore_axis_name="s")
    fn = pl.kernel(
        functools.partial(_gather_body, sl * NGRP_SL),
        out_type=(jax.ShapeDtypeStruct((ESL, D), jnp.float32),
                  jax.ShapeDtypeStruct((ESL, D), jnp.float32)),
        mesh=mesh,
        scratch_types=[
            pltpu.VMEM((RPG, ROW), jnp.int32),
            pltpu.VMEM((RPG, ROW), jnp.int32),
            pltpu.VMEM((RPG, ROW), jnp.int32),
            pltpu.VMEM((RPG, ROW), jnp.int32),
            pltpu.VMEM((GRP, D), jnp.float32),
            pltpu.VMEM((GRP, D), jnp.float32),
            pltpu.VMEM((GRP, D), jnp.float32),
            pltpu.VMEM((GRP, D), jnp.float32),
            pltpu.SemaphoreType.DMA,
            pltpu.SemaphoreType.DMA,
            pltpu.SemaphoreType.DMA,
            pltpu.SemaphoreType.DMA,
            pltpu.SemaphoreType.DMA,
            pltpu.SemaphoreType.DMA,
        ],
        compiler_params=pltpu.CompilerParams(use_tc_tiling_on_sc=False),
    )
    return fn(h, s2d, e2d)


EBR = 1000


def _edge_body(hs_ref, he_ref, W1a, W1b, b1, g1, be1, W2e, b2, g2, be2,
               B32, B17, P16, m16, s16, o_ref):
    Z = (jnp.dot(hs_ref[...], W1a[...], preferred_element_type=jnp.float32)
         + jnp.dot(he_ref[...], W1b[...], preferred_element_type=jnp.float32)
         + b1[...])
    mu = jnp.dot(Z, B32[...], preferred_element_type=jnp.float32)
    d = Z - mu
    var = jnp.dot(d * d, B32[...], preferred_element_type=jnp.float32)
    A = d * lax.rsqrt(var + EPS_LN) * g1[...] + be1[...]
    A = A * jax.nn.sigmoid(A)
    Z2 = jnp.dot(A, W2e[...], preferred_element_type=jnp.float32) + b2[...]
    mu2 = jnp.dot(Z2, B17[...], preferred_element_type=jnp.float32)
    d2 = Z2 - mu2
    var2 = jnp.dot(d2 * d2, B17[...], preferred_element_type=jnp.float32)
    Y = d2 * lax.rsqrt(var2 + EPS_LN) * g2[...] + be2[...]
    Y = Y * jax.nn.sigmoid(Y)
    expw = jnp.exp(jnp.dot(Y, P16[...], preferred_element_type=jnp.float32))
    o_ref[...] = (Y * m16[...] + s16[...]) * expw


def _edge_setup(p):
    (W1, b1, g1, be1), (W2, b2, g2, be2) = p
    I8 = jnp.eye(8, dtype=jnp.float32)
    pos = jnp.arange(32)
    W2p = jnp.pad(W2, ((0, 0), (0, 15)))
    seg17 = jnp.where(pos < 17, 1.0 / 17.0, 0.0)[:, None] * jnp.ones((1, 32))
    p16 = jnp.where(pos == 16, 1.0, 0.0)[:, None] * jnp.ones((1, 32))
    pad17 = lambda v: jnp.pad(v, (0, 15))
    return [
        jnp.kron(I8, W1[:D]), jnp.kron(I8, W1[D:]),
        jnp.tile(b1, 8).reshape(1, -1), jnp.tile(g1, 8).reshape(1, -1),
        jnp.tile(be1, 8).reshape(1, -1),
        jnp.kron(I8, W2p),
        jnp.tile(pad17(b2), 8).reshape(1, -1),
        jnp.tile(pad17(g2), 8).reshape(1, -1),
        jnp.tile(pad17(be2), 8).reshape(1, -1),
        jnp.kron(I8, jnp.full((32, 32), 1.0 / 32.0)),
        jnp.kron(I8, seg17),
        jnp.kron(I8, p16),
        jnp.tile(jnp.where(pos < D, 1.0, 0.0), 8).reshape(1, -1),
        jnp.tile(jnp.where(pos == D, 1.0, 0.0), 8).reshape(1, -1),
    ]


def _run_edge(hs_pk, he_pk, params, interpret=False):
    in_specs = [pl.BlockSpec((EBR, 128), lambda i: (i, 0)),
                pl.BlockSpec((EBR, 128), lambda i: (i, 0))]
    in_specs += [pl.BlockSpec(w.shape, lambda i: (0,) * w.ndim) for w in params]
    n_pk = hs_pk.shape[0]
    return pl.pallas_call(
        _edge_body,
        grid=(n_pk // EBR,),
        in_specs=in_specs,
        out_specs=pl.BlockSpec((EBR, 256), lambda i: (i, 0)),
        out_shape=jax.ShapeDtypeStruct((n_pk, 256), jnp.float32),
        interpret=interpret,
    )(hs_pk, he_pk, *params)


def _scatter_body(ew0_hbm, ew1_hbm, e2d_hbm, zeros_hbm, acc_hbm, *refs):
    (idx_a, idx_b, buf_a, buf_b, sem_a, sem_b, sem_z, acc) = refs
    c = lax.axis_index("c")
    s = lax.axis_index("s")
    w = s * 2 + c

    zcp = pltpu.make_async_copy(
        zeros_hbm.at[pl.ds(s * _NROWS_TILE, _NROWS_TILE)],
        acc.at[pl.ds(s * _NROWS_TILE, _NROWS_TILE)], sem_z)
    zcp.start()
    zcp.wait()
    plsc.subcore_barrier()

    def drain(ew_hbm, w_eff, sl):
        g0 = w_eff * _GRP_S_BASE + jnp.minimum(w_eff, _GRP_S_EXTRA)
        cnt = _GRP_S_BASE + jnp.where(w_eff < _GRP_S_EXTRA, 1, 0)
        idx_base = sl * NGRP_S_SL

        def fetch(g, idx, buf, sem):
            return (pltpu.make_async_copy(
                        ew_hbm.at[pl.ds(GRP_S * g, GRP_S)], buf, sem),
                    pltpu.make_async_copy(
                        e2d_hbm.at[pl.ds(RPG_S * (idx_base + g), RPG_S)],
                        idx, sem))

        def scat(idx, buf):
            for j in range(RPG_S):
                pltpu.sync_copy(buf.at[pl.ds(j * ROW, ROW)],
                                acc.at[idx.at[j]], add=True)

        for cp in fetch(g0, idx_a, buf_a, sem_a):
            cp.start()

        def body(p, carry):
            ga = g0 + 2 * p
            gb = ga + 1

            @pl.when(2 * p < cnt)
            def _():
                for cp in fetch(ga, idx_a, buf_a, sem_a):
                    cp.wait()

                @pl.when(2 * p + 1 < cnt)
                def _():
                    for cp in fetch(gb, idx_b, buf_b, sem_b):
                        cp.start()
                scat(idx_a, buf_a)

                @pl.when(2 * p + 2 < cnt)
                def _():
                    for cp in fetch(ga + 2, idx_a, buf_a, sem_a):
                        cp.start()

            @pl.when(2 * p + 1 < cnt)
            def _():
                for cp in fetch(gb, idx_b, buf_b, sem_b):
                    cp.wait()
                scat(idx_b, buf_b)
            return carry

        lax.fori_loop(0, _NPAIR_S, body, 0)

    @pl.when(w < _NW_S)
    def _():
        drain(ew0_hbm, w, 0)

    @pl.when(w >= _NW_S)
    def _():
        drain(ew1_hbm, w - _NW_S, 1)

    plsc.subcore_barrier()
    pltpu.sync_copy(acc.at[pl.ds(s * _NROWS_TILE, _NROWS_TILE)],
                    acc_hbm.at[c, pl.ds(s * _NROWS_TILE, _NROWS_TILE)])


def _run_scatter(ew0, ew1, e2d, zeros):
    mesh = plsc.VectorSubcoreMesh(core_axis_name="c", subcore_axis_name="s")
    fn = pl.kernel(
        _scatter_body,
        out_type=jax.ShapeDtypeStruct((2, N, 32), jnp.float32),
        mesh=mesh,
        scratch_types=[
            pltpu.VMEM((RPG_S, ROW), jnp.int32),
            pltpu.VMEM((RPG_S, ROW), jnp.int32),
            pltpu.VMEM((GRP_S, 32), jnp.float32),
            pltpu.VMEM((GRP_S, 32), jnp.float32),
            pltpu.SemaphoreType.DMA,
            pltpu.SemaphoreType.DMA,
            pltpu.SemaphoreType.DMA,
            pltpu.VMEM_SHARED((N, 32), jnp.float32),
        ],
        compiler_params=pltpu.CompilerParams(use_tc_tiling_on_sc=False),
    )
    return fn(ew0, ew1, e2d, zeros)


def _node_body(h_ref, acc_ref,
               Wn1a, Wn1b, bn1, gn1, ben1, Wn2, bn2, gn2, ben2,
               Wd1, bd1, gd1, bed1, Wd2, bd2, gd2, bed2, o_ref):
    accs = acc_ref[0] + acc_ref[1]
    agg = accs[:, :D] / (accs[:, D:D + 1] + 1e-16)
    Z = (jnp.dot(h_ref[...], Wn1a[...], preferred_element_type=jnp.float32)
         + jnp.dot(agg, Wn1b[...], preferred_element_type=jnp.float32)
         + bn1[...])
    mu = jnp.mean(Z, axis=-1, keepdims=True)
    Zc = Z - mu
    var = jnp.mean(Zc * Zc, axis=-1, keepdims=True)
    A = Zc * lax.rsqrt(var + EPS_LN) * gn1[...] + ben1[...]
    A = A * jax.nn.sigmoid(A)
    h3 = _layer(A, Wn2[...], bn2[...], gn2[...], ben2[...], 'silu')
    B1 = _layer(h3, Wd1[...], bd1[...], gd1[...], bed1[...], 'silu')
    T = _layer(B1, Wd2[...], bd2[...], gd2[...], bed2[...], 'tanh')
    nrm = jnp.sqrt(jnp.sum(T * T, axis=-1, keepdims=True)) + 1e-12
    o_ref[...] = T / nrm


def _run_node(h, acc, node_p, dec1_p, interpret=False):
    (Wn1, bn1, gn1, ben1), (Wn2, bn2, gn2, ben2) = node_p
    (Wd1, bd1, gd1, bed1), (Wd2, bd2, gd2, bed2) = dec1_p
    params = [Wn1[:D], Wn1[D:], bn1.reshape(1, -1), gn1.reshape(1, -1),
              ben1.reshape(1, -1), Wn2, bn2.reshape(1, -1), gn2.reshape(1, -1),
              ben2.reshape(1, -1),
              Wd1, bd1.reshape(1, -1), gd1.reshape(1, -1), bed1.reshape(1, -1),
              Wd2, bd2.reshape(1, -1), gd2.reshape(1, -1), bed2.reshape(1, -1)]
    in_specs = [pl.BlockSpec((NB, D), lambda i: (i, 0)),
                pl.BlockSpec((2, NB, 32), lambda i: (0, i, 0))]
    in_specs += [pl.BlockSpec(w.shape, lambda i: (0, 0)) for w in params]
    return pl.pallas_call(
        _node_body,
        grid=(N // NB,),
        in_specs=in_specs,
        out_specs=pl.BlockSpec((NB, 8), lambda i: (i, 0)),
        out_shape=jax.ShapeDtypeStruct((N, 8), jnp.float32),
        interpret=interpret,
    )(h, acc, *params)


def kernel(x, start, end, enc_p, net0_p, edge_p, node_p, dec0_p, dec1_p):
    del net0_p, dec0_p
    s2d = start.reshape(E // ROW, ROW)
    e2d = end.reshape(E // ROW, ROW)
    zeros = jnp.zeros((N, 32), jnp.float32)
    h = _run_enc(x, enc_p)
    eparams = _edge_setup(edge_p)
    hs0, he0 = _run_gather(h, s2d, e2d, 0)
    hs1, he1 = _run_gather(h, s2d, e2d, 1)
    ew0 = _run_edge(hs0.reshape(ESL // 8, 128), he0.reshape(ESL // 8, 128),
                    eparams)
    ew1 = _run_edge(hs1.reshape(ESL // 8, 128), he1.reshape(ESL // 8, 128),
                    eparams)
    acc = _run_scatter(ew0.reshape(ESL, 32), ew1.reshape(ESL, 32), e2d, zeros)
    return _run_node(h, acc, node_p, dec1_p)


# speedup vs baseline: 1.0694x; 1.0407x over previous
import functools

import jax
import jax.numpy as jnp
from jax import lax
from jax.experimental import pallas as pl
from jax.experimental.pallas import tpu as pltpu, tpu_sc as plsc

N = 50000
E = 1600000
D = 16
ROW = 128
RPG = 10
GRP = RPG * ROW
NGRP = E // GRP
NW = 32
NB = 5000
EPS_LN = 1e-5

NSL = 2
ESL = E // NSL
NGRP_SL = ESL // GRP
_GRP_BASE = NGRP_SL // NW
_GRP_EXTRA = NGRP_SL % NW
_NPAIR = (_GRP_BASE + 2) // 2
_NROWS_TILE = N // 16

RPG_S = 2
GRP_S = RPG_S * ROW
NGRP_S_SL = ESL // GRP_S
_NW_S = NW // NSL
_GRP_S_BASE = NGRP_S_SL // _NW_S
_GRP_S_EXTRA = NGRP_S_SL % _NW_S
_NPAIR_S = (_GRP_S_BASE + 2) // 2


def _layer(X, W, b, g, be, act):
    Z = jnp.dot(X, W, preferred_element_type=jnp.float32) + b
    mu = jnp.mean(Z, axis=-1, keepdims=True)
    Zc = Z - mu
    var = jnp.mean(Zc * Zc, axis=-1, keepdims=True)
    Y = Zc * lax.rsqrt(var + EPS_LN) * g + be
    if act == 'silu':
        Y = Y * jax.nn.sigmoid(Y)
    elif act == 'tanh':
        Y = jnp.tanh(Y)
    return Y


def _enc_body(x_ref, W1, b1, g1, be1, W2, b2, g2, be2, o_ref):
    X = x_ref[...]
    A = _layer(X, W1[...], b1[...], g1[...], be1[...], 'silu')
    o_ref[...] = _layer(A, W2[...], b2[...], g2[...], be2[...], 'silu')


def _run_enc(x, p, interpret=False):
    (W1, b1, g1, be1), (W2, b2, g2, be2) = p
    params = [W1, b1.reshape(1, -1), g1.reshape(1, -1), be1.reshape(1, -1),
              W2, b2.reshape(1, -1), g2.reshape(1, -1), be2.reshape(1, -1)]
    in_specs = [pl.BlockSpec((NB, x.shape[1]), lambda i: (i, 0))]
    in_specs += [pl.BlockSpec(w.shape, lambda i: (0, 0)) for w in params]
    return pl.pallas_call(
        _enc_body,
        grid=(N // NB,),
        in_specs=in_specs,
        out_specs=pl.BlockSpec((NB, D), lambda i: (i, 0)),
        out_shape=jax.ShapeDtypeStruct((N, D), jnp.float32),
        interpret=interpret,
    )(x, *params)


def _gather_body(sl_ofs, h_hbm, s2d_hbm, e2d_hbm, hs_hbm, he_hbm, *refs):
    (idx_sa, idx_ea, idx_sb, idx_eb, buf_sa, buf_ea, buf_sb, buf_eb,
     sem_ia, sem_ib, sem_ga, sem_gb, sem_wa, sem_wb) = refs
    c = lax.axis_index("c")
    s = lax.axis_index("s")
    w = s * 2 + c
    g0 = w * _GRP_BASE + jnp.minimum(w, _GRP_EXTRA)
    cnt = _GRP_BASE + jnp.where(w < _GRP_EXTRA, 1, 0)

    def idx_cp(g, idx_s, idx_e, sem):
        a = pltpu.make_async_copy(
            s2d_hbm.at[pl.ds(RPG * (sl_ofs + g), RPG)], idx_s, sem)
        b = pltpu.make_async_copy(
            e2d_hbm.at[pl.ds(RPG * (sl_ofs + g), RPG)], idx_e, sem)
        return a, b

    def gath(idx_s, idx_e, buf_s, buf_e, sem):
        return ([pltpu.make_async_copy(h_hbm.at[idx_s.at[j]],
                                       buf_s.at[pl.ds(j * ROW, ROW)], sem)
                 for j in range(RPG)] +
                [pltpu.make_async_copy(h_hbm.at[idx_e.at[j]],
                                       buf_e.at[pl.ds(j * ROW, ROW)], sem)
                 for j in range(RPG)])

    def wb(g, buf_s, buf_e, sem):
        a = pltpu.make_async_copy(buf_s, hs_hbm.at[pl.ds(GRP * g, GRP)], sem)
        b = pltpu.make_async_copy(buf_e, he_hbm.at[pl.ds(GRP * g, GRP)], sem)
        return a, b

    @pl.when(cnt > 0)
    def _():
        for cp in idx_cp(g0, idx_sa, idx_ea, sem_ia):
            cp.start()

    def body(p, carry):
        ga = g0 + 2 * p
        gb = ga + 1
        na = 2 * p
        nb = 2 * p + 1

        @pl.when(nb < cnt)
        def _():
            for cp in idx_cp(gb, idx_sb, idx_eb, sem_ib):
                cp.start()

        @pl.when(na < cnt)
        def _():
            for cp in idx_cp(ga, idx_sa, idx_ea, sem_ia):
                cp.wait()

            @pl.when(p > 0)
            def _():
                for cp in wb(ga, buf_sa, buf_ea, sem_wa):
                    cp.wait()
            for cp in gath(idx_sa, idx_ea, buf_sa, buf_ea, sem_ga):
                cp.start()

        @pl.when(na < cnt)
        def _():
            for cp in gath(idx_sa, idx_ea, buf_sa, buf_ea, sem_ga):
                cp.wait()
            for cp in wb(ga, buf_sa, buf_ea, sem_wa):
                cp.start()

        @pl.when(na + 2 < cnt)
        def _():
            for cp in idx_cp(ga + 2, idx_sa, idx_ea, sem_ia):
                cp.start()

        @pl.when(nb < cnt)
        def _():
            for cp in idx_cp(gb, idx_sb, idx_eb, sem_ib):
                cp.wait()

            @pl.when(p > 0)
            def _():
                for cp in wb(gb, buf_sb, buf_eb, sem_wb):
                    cp.wait()
            for cp in gath(idx_sb, idx_eb, buf_sb, buf_eb, sem_gb):
                cp.start()
            for cp in gath(idx_sb, idx_eb, buf_sb, buf_eb, sem_gb):
                cp.wait()
            for cp in wb(gb, buf_sb, buf_eb, sem_wb):
                cp.start()

        return carry

    lax.fori_loop(0, _NPAIR, body, 0)

    @pl.when(cnt > 0)
    def _():
        for cp in wb(g0, buf_sa, buf_ea, sem_wa):
            cp.wait()

    @pl.when(cnt > 1)
    def _():
        for cp in wb(g0 + 1, buf_sb, buf_eb, sem_wb):
            cp.wait()


def _run_gather(h, s2d, e2d, sl):
    mesh = plsc.VectorSubcoreMesh(core_axis_name="c", subcore_axis_name="s")
    fn = pl.kernel(
        functools.partial(_gather_body, sl * NGRP_SL),
        out_type=(jax.ShapeDtypeStruct((ESL, D), jnp.float32),
                  jax.ShapeDtypeStruct((ESL, D), jnp.float32)),
        mesh=mesh,
        scratch_types=[
            pltpu.VMEM((RPG, ROW), jnp.int32),
            pltpu.VMEM((RPG, ROW), jnp.int32),
            pltpu.VMEM((RPG, ROW), jnp.int32),
            pltpu.VMEM((RPG, ROW), jnp.int32),
            pltpu.VMEM((GRP, D), jnp.float32),
            pltpu.VMEM((GRP, D), jnp.float32),
            pltpu.VMEM((GRP, D), jnp.float32),
            pltpu.VMEM((GRP, D), jnp.float32),
            pltpu.SemaphoreType.DMA,
            pltpu.SemaphoreType.DMA,
            pltpu.SemaphoreType.DMA,
            pltpu.SemaphoreType.DMA,
            pltpu.SemaphoreType.DMA,
            pltpu.SemaphoreType.DMA,
        ],
        compiler_params=pltpu.CompilerParams(use_tc_tiling_on_sc=False),
    )
    return fn(h, s2d, e2d)


EBR = 2000


def _edge_body(hs_ref, he_ref, W1a, W1b, b1, g1, be1, W2e, b2, g2, be2,
               B32, B17, P16, m16, s16, o_ref):
    Z = (jnp.dot(hs_ref[...], W1a[...], preferred_element_type=jnp.float32)
         + jnp.dot(he_ref[...], W1b[...], preferred_element_type=jnp.float32)
         + b1[...])
    mu = jnp.dot(Z, B32[...], preferred_element_type=jnp.float32)
    d = Z - mu
    var = jnp.dot(d * d, B32[...], preferred_element_type=jnp.float32)
    A = d * lax.rsqrt(var + EPS_LN) * g1[...] + be1[...]
    A = A * jax.nn.sigmoid(A)
    Z2 = jnp.dot(A, W2e[...], preferred_element_type=jnp.float32) + b2[...]
    mu2 = jnp.dot(Z2, B17[...], preferred_element_type=jnp.float32)
    d2 = Z2 - mu2
    var2 = jnp.dot(d2 * d2, B17[...], preferred_element_type=jnp.float32)
    Y = d2 * lax.rsqrt(var2 + EPS_LN) * g2[...] + be2[...]
    Y = Y * jax.nn.sigmoid(Y)
    expw = jnp.exp(jnp.dot(Y, P16[...], preferred_element_type=jnp.float32))
    o_ref[...] = (Y * m16[...] + s16[...]) * expw


def _edge_setup(p):
    (W1, b1, g1, be1), (W2, b2, g2, be2) = p
    I8 = jnp.eye(8, dtype=jnp.float32)
    pos = jnp.arange(32)
    W2p = jnp.pad(W2, ((0, 0), (0, 15)))
    seg17 = jnp.where(pos < 17, 1.0 / 17.0, 0.0)[:, None] * jnp.ones((1, 32))
    p16 = jnp.where(pos == 16, 1.0, 0.0)[:, None] * jnp.ones((1, 32))
    pad17 = lambda v: jnp.pad(v, (0, 15))
    return [
        jnp.kron(I8, W1[:D]), jnp.kron(I8, W1[D:]),
        jnp.tile(b1, 8).reshape(1, -1), jnp.tile(g1, 8).reshape(1, -1),
        jnp.tile(be1, 8).reshape(1, -1),
        jnp.kron(I8, W2p),
        jnp.tile(pad17(b2), 8).reshape(1, -1),
        jnp.tile(pad17(g2), 8).reshape(1, -1),
        jnp.tile(pad17(be2), 8).reshape(1, -1),
        jnp.kron(I8, jnp.full((32, 32), 1.0 / 32.0)),
        jnp.kron(I8, seg17),
        jnp.kron(I8, p16),
        jnp.tile(jnp.where(pos < D, 1.0, 0.0), 8).reshape(1, -1),
        jnp.tile(jnp.where(pos == D, 1.0, 0.0), 8).reshape(1, -1),
    ]


def _run_edge(hs_pk, he_pk, params, interpret=False):
    in_specs = [pl.BlockSpec((EBR, 128), lambda i: (i, 0)),
                pl.BlockSpec((EBR, 128), lambda i: (i, 0))]
    in_specs += [pl.BlockSpec(w.shape, lambda i: (0,) * w.ndim) for w in params]
    n_pk = hs_pk.shape[0]
    return pl.pallas_call(
        _edge_body,
        grid=(n_pk // EBR,),
        in_specs=in_specs,
        out_specs=pl.BlockSpec((EBR, 256), lambda i: (i, 0)),
        out_shape=jax.ShapeDtypeStruct((n_pk, 256), jnp.float32),
        interpret=interpret,
    )(hs_pk, he_pk, *params)


def _scatter_body(ew0_hbm, ew1_hbm, e2d_hbm, zeros_hbm, acc_hbm, *refs):
    (idx_a, idx_b, buf_a, buf_b, sem_a, sem_b, sem_z, acc) = refs
    c = lax.axis_index("c")
    s = lax.axis_index("s")
    w = s * 2 + c

    zcp = pltpu.make_async_copy(
        zeros_hbm.at[pl.ds(s * _NROWS_TILE, _NROWS_TILE)],
        acc.at[pl.ds(s * _NROWS_TILE, _NROWS_TILE)], sem_z)
    zcp.start()
    zcp.wait()
    plsc.subcore_barrier()

    def drain(ew_hbm, w_eff, sl):
        g0 = w_eff * _GRP_S_BASE + jnp.minimum(w_eff, _GRP_S_EXTRA)
        cnt = _GRP_S_BASE + jnp.where(w_eff < _GRP_S_EXTRA, 1, 0)
        idx_base = sl * NGRP_S_SL

        def fetch(g, idx, buf, sem):
            return (pltpu.make_async_copy(
                        ew_hbm.at[pl.ds(GRP_S * g, GRP_S)], buf, sem),
                    pltpu.make_async_copy(
                        e2d_hbm.at[pl.ds(RPG_S * (idx_base + g), RPG_S)],
                        idx, sem))

        def scat(idx, buf):
            for j in range(RPG_S):
                pltpu.sync_copy(buf.at[pl.ds(j * ROW, ROW)],
                                acc.at[idx.at[j]], add=True)

        for cp in fetch(g0, idx_a, buf_a, sem_a):
            cp.start()

        def body(p, carry):
            ga = g0 + 2 * p
            gb = ga + 1

            @pl.when(2 * p < cnt)
            def _():
                for cp in fetch(ga, idx_a, buf_a, sem_a):
                    cp.wait()

                @pl.when(2 * p + 1 < cnt)
                def _():
                    for cp in fetch(gb, idx_b, buf_b, sem_b):
                        cp.start()
                scat(idx_a, buf_a)

                @pl.when(2 * p + 2 < cnt)
                def _():
                    for cp in fetch(ga + 2, idx_a, buf_a, sem_a):
                        cp.start()

            @pl.when(2 * p + 1 < cnt)
            def _():
                for cp in fetch(gb, idx_b, buf_b, sem_b):
                    cp.wait()
                scat(idx_b, buf_b)
            return carry

        lax.fori_loop(0, _NPAIR_S, body, 0)

    @pl.when(w < _NW_S)
    def _():
        drain(ew0_hbm, w, 0)

    @pl.when(w >= _NW_S)
    def _():
        drain(ew1_hbm, w - _NW_S, 1)

    plsc.subcore_barrier()
    pltpu.sync_copy(acc.at[pl.ds(s * _NROWS_TILE, _NROWS_TILE)],
                    acc_hbm.at[c, pl.ds(s * _NROWS_TILE, _NROWS_TILE)])


def _run_scatter(ew0, ew1, e2d, zeros):
    mesh = plsc.VectorSubcoreMesh(core_axis_name="c", subcore_axis_name="s")
    fn = pl.kernel(
        _scatter_body,
        out_type=jax.ShapeDtypeStruct((2, N, 32), jnp.float32),
        mesh=mesh,
        scratch_types=[
            pltpu.VMEM((RPG_S, ROW), jnp.int32),
            pltpu.VMEM((RPG_S, ROW), jnp.int32),
            pltpu.VMEM((GRP_S, 32), jnp.float32),
            pltpu.VMEM((GRP_S, 32), jnp.float32),
            pltpu.SemaphoreType.DMA,
            pltpu.SemaphoreType.DMA,
            pltpu.SemaphoreType.DMA,
            pltpu.VMEM_SHARED((N, 32), jnp.float32),
        ],
        compiler_params=pltpu.CompilerParams(use_tc_tiling_on_sc=False),
    )
    return fn(ew0, ew1, e2d, zeros)


def _node_body(h_ref, acc_ref,
               Wn1a, Wn1b, bn1, gn1, ben1, Wn2, bn2, gn2, ben2,
               Wd1, bd1, gd1, bed1, Wd2, bd2, gd2, bed2, o_ref):
    accs = acc_ref[0] + acc_ref[1]
    agg = accs[:, :D] / (accs[:, D:D + 1] + 1e-16)
    Z = (jnp.dot(h_ref[...], Wn1a[...], preferred_element_type=jnp.float32)
         + jnp.dot(agg, Wn1b[...], preferred_element_type=jnp.float32)
         + bn1[...])
    mu = jnp.mean(Z, axis=-1, keepdims=True)
    Zc = Z - mu
    var = jnp.mean(Zc * Zc, axis=-1, keepdims=True)
    A = Zc * lax.rsqrt(var + EPS_LN) * gn1[...] + ben1[...]
    A = A * jax.nn.sigmoid(A)
    h3 = _layer(A, Wn2[...], bn2[...], gn2[...], ben2[...], 'silu')
    B1 = _layer(h3, Wd1[...], bd1[...], gd1[...], bed1[...], 'silu')
    T = _layer(B1, Wd2[...], bd2[...], gd2[...], bed2[...], 'tanh')
    nrm = jnp.sqrt(jnp.sum(T * T, axis=-1, keepdims=True)) + 1e-12
    o_ref[...] = T / nrm


def _run_node(h, acc, node_p, dec1_p, interpret=False):
    (Wn1, bn1, gn1, ben1), (Wn2, bn2, gn2, ben2) = node_p
    (Wd1, bd1, gd1, bed1), (Wd2, bd2, gd2, bed2) = dec1_p
    params = [Wn1[:D], Wn1[D:], bn1.reshape(1, -1), gn1.reshape(1, -1),
              ben1.reshape(1, -1), Wn2, bn2.reshape(1, -1), gn2.reshape(1, -1),
              ben2.reshape(1, -1),
              Wd1, bd1.reshape(1, -1), gd1.reshape(1, -1), bed1.reshape(1, -1),
              Wd2, bd2.reshape(1, -1), gd2.reshape(1, -1), bed2.reshape(1, -1)]
    in_specs = [pl.BlockSpec((NB, D), lambda i: (i, 0)),
                pl.BlockSpec((2, NB, 32), lambda i: (0, i, 0))]
    in_specs += [pl.BlockSpec(w.shape, lambda i: (0, 0)) for w in params]
    return pl.pallas_call(
        _node_body,
        grid=(N // NB,),
        in_specs=in_specs,
        out_specs=pl.BlockSpec((NB, 8), lambda i: (i, 0)),
        out_shape=jax.ShapeDtypeStruct((N, 8), jnp.float32),
        interpret=interpret,
    )(h, acc, *params)


def kernel(x, start, end, enc_p, net0_p, edge_p, node_p, dec0_p, dec1_p):
    del net0_p, dec0_p
    s2d = start.reshape(E // ROW, ROW)
    e2d = end.reshape(E // ROW, ROW)
    zeros = jnp.zeros((N, 32), jnp.float32)
    h = _run_enc(x, enc_p)
    eparams = _edge_setup(edge_p)
    hs0, he0 = _run_gather(h, s2d, e2d, 0)
    hs1, he1 = _run_gather(h, s2d, e2d, 1)
    ew0 = _run_edge(hs0.reshape(ESL // 8, 128), he0.reshape(ESL // 8, 128),
                    eparams)
    ew1 = _run_edge(hs1.reshape(ESL // 8, 128), he1.reshape(ESL // 8, 128),
                    eparams)
    acc = _run_scatter(ew0.reshape(ESL, 32), ew1.reshape(ESL, 32), e2d, zeros)
    return _run_node(h, acc, node_p, dec1_p)
